# baseline (speedup 1.0000x reference)
import jax
import jax.numpy as jnp
from jax.experimental import pallas as pl
from jax.experimental.pallas import tpu as pltpu

_L = 64
_D = 2048
_B = 4
_S = 2048

_CHUNK = 1024
_CPB = _S // _CHUNK
_NCHUNKS = _B * _CPB
_NBUF = 7
_LOOK = 6


def _chunk_src_dst(c, emb_ref, out_ref):
    b, j = divmod(c, _CPB)
    src = emb_ref.at[b, pl.ds(j * _CHUNK, _CHUNK), :]
    dst = out_ref.at[b, pl.ds(_L + j * _CHUNK, _CHUNK), :]
    return src, dst


def _body(emb_ref, prompt_ref, out_ref, buf, in_sems, out_sems, psem):
    def in_dma(c):
        src, _ = _chunk_src_dst(c, emb_ref, out_ref)
        return pltpu.make_async_copy(src, buf.at[c % _NBUF], in_sems.at[c % _NBUF])

    def out_dma(c):
        _, dst = _chunk_src_dst(c, emb_ref, out_ref)
        return pltpu.make_async_copy(buf.at[c % _NBUF], dst, out_sems.at[c % _NBUF])

    for b in range(_B):
        pltpu.make_async_copy(prompt_ref, out_ref.at[b, pl.ds(0, _L), :], psem).start()

    for c in range(_LOOK):
        in_dma(c).start()

    for c in range(_NCHUNKS):
        in_dma(c).wait()
        out_dma(c).start()
        nxt = c + _LOOK
        if nxt < _NCHUNKS:
            if nxt >= _NBUF:
                out_dma(nxt - _NBUF).wait()
            in_dma(nxt).start()

    for c in range(max(_NCHUNKS - _NBUF, 0), _NCHUNKS):
        out_dma(c).wait()
    for b in range(_B):
        pltpu.make_async_copy(prompt_ref, out_ref.at[b, pl.ds(0, _L), :], psem).wait()


@jax.jit
def kernel(embedded_input, prompt_table):
    out = pl.pallas_call(
        _body,
        out_shape=jax.ShapeDtypeStruct((_B, _L + _S, _D), jnp.float32),
        in_specs=[
            pl.BlockSpec(memory_space=pltpu.MemorySpace.HBM),
            pl.BlockSpec(memory_space=pltpu.MemorySpace.HBM),
        ],
        out_specs=pl.BlockSpec(memory_space=pltpu.MemorySpace.HBM),
        scratch_shapes=[
            pltpu.VMEM((_NBUF, _CHUNK, _D), jnp.float32),
            pltpu.SemaphoreType.DMA((_NBUF,)),
            pltpu.SemaphoreType.DMA((_NBUF,)),
            pltpu.SemaphoreType.DMA,
        ],
    )(embedded_input, prompt_table)
    mask = jnp.ones((_B, _L), dtype=jnp.float32)
    return (out, mask)


# speedup vs baseline: 1.5960x; 1.5960x over previous
import jax
import jax.numpy as jnp
from jax.experimental import pallas as pl
from jax.experimental.pallas import tpu as pltpu

_L = 64
_D = 2048
_B = 4
_S = 2048

_CHUNK = 1024
_CPB = _S // _CHUNK
_NCHUNKS = _B * _CPB
_NBUF = 7
_LOOK = 6


def _chunk_src_dst(c, emb_ref, out_ref):
    b, j = divmod(c, _CPB)
    src = emb_ref.at[b, pl.ds(j * _CHUNK, _CHUNK), :]
    dst = out_ref.at[b, pl.ds(_L + j * _CHUNK, _CHUNK), :]
    return src, dst


def _body(emb_ref, prompt_ref, out_ref, buf, pbuf, in_sems, out_sems, psem):
    def in_dma(c):
        src, _ = _chunk_src_dst(c, emb_ref, out_ref)
        return pltpu.make_async_copy(src, buf.at[c % _NBUF], in_sems.at[c % _NBUF])

    def out_dma(c):
        _, dst = _chunk_src_dst(c, emb_ref, out_ref)
        return pltpu.make_async_copy(buf.at[c % _NBUF], dst, out_sems.at[c % _NBUF])

    pltpu.make_async_copy(prompt_ref, pbuf, psem).start()
    for c in range(_LOOK):
        in_dma(c).start()
    pltpu.make_async_copy(prompt_ref, pbuf, psem).wait()
    for b in range(_B):
        pltpu.make_async_copy(pbuf, out_ref.at[b, pl.ds(0, _L), :], psem).start()

    for c in range(_NCHUNKS):
        in_dma(c).wait()
        out_dma(c).start()
        nxt = c + _LOOK
        if nxt < _NCHUNKS:
            if nxt >= _NBUF:
                out_dma(nxt - _NBUF).wait()
            in_dma(nxt).start()

    for c in range(max(_NCHUNKS - _NBUF, 0), _NCHUNKS):
        out_dma(c).wait()
    for b in range(_B):
        pltpu.make_async_copy(pbuf, out_ref.at[b, pl.ds(0, _L), :], psem).wait()


@jax.jit
def kernel(embedded_input, prompt_table):
    out = pl.pallas_call(
        _body,
        out_shape=jax.ShapeDtypeStruct((_B, _L + _S, _D), jnp.float32),
        in_specs=[
            pl.BlockSpec(memory_space=pltpu.MemorySpace.HBM),
            pl.BlockSpec(memory_space=pltpu.MemorySpace.HBM),
        ],
        out_specs=pl.BlockSpec(memory_space=pltpu.MemorySpace.HBM),
        scratch_shapes=[
            pltpu.VMEM((_NBUF, _CHUNK, _D), jnp.float32),
            pltpu.VMEM((_L, _D), jnp.float32),
            pltpu.SemaphoreType.DMA((_NBUF,)),
            pltpu.SemaphoreType.DMA((_NBUF,)),
            pltpu.SemaphoreType.DMA,
        ],
    )(embedded_input, prompt_table)
    mask = jnp.ones((_B, _L), dtype=jnp.float32)
    return (out, mask)
